# baseline (speedup 1.0000x reference)
import functools

import jax
import jax.numpy as jnp
from jax import lax
from jax.experimental import pallas as pl
from jax.experimental.pallas import tpu as pltpu
from jax.experimental.pallas import tpu_sc as plsc

VOCAB = 100000
EMBED_DIM = 32
CONTEXT = 20
PAD_CTX = 32
FAN_IN = CONTEXT * EMBED_DIM
NE = FAN_IN // 16

NW = 32
CH = 32
NCH = 30
S = CH * NCH
R_SC = NW * S
R0 = VOCAB - R_SC

VBS = 2048
NFULL = R0 // VBS
TAIL = R0 - NFULL * VBS
NB = NFULL + 1
DEPTH = 4


def _sc_matvec(idx_pad, table, W):
    mesh = plsc.VectorSubcoreMesh(core_axis_name="c", subcore_axis_name="s")

    @functools.partial(
        pl.kernel,
        mesh=mesh,
        out_type=jax.ShapeDtypeStruct((R_SC,), jnp.float32),
        scratch_types=[
            pltpu.VMEM((PAD_CTX,), jnp.int32),
            pltpu.VMEM((PAD_CTX, EMBED_DIM), jnp.float32),
            pltpu.VMEM((2, CH, FAN_IN), jnp.float32),
            pltpu.VMEM((2, CH), jnp.float32),
            pltpu.SemaphoreType.DMA,
            pltpu.SemaphoreType.DMA((2,)),
            pltpu.SemaphoreType.DMA((2,)),
        ],
        compiler_params=pltpu.CompilerParams(
            use_tc_tiling_on_sc=False, needs_layout_passes=False),
    )
    def k(idx_hbm, table_hbm, w_hbm, out_hbm,
          idx_v, rows_v, w_ring, y_ring, gsem, wsems, osems):
        wid = lax.axis_index("s") * 2 + lax.axis_index("c")
        wbase = R0 + wid * S
        obase = wid * S

        pltpu.sync_copy(idx_hbm, idx_v)
        pltpu.async_copy(table_hbm.at[idx_v], rows_v, gsem).wait()
        evecs = [rows_v[t // 2, pl.ds((t % 2) * 16, 16)] for t in range(NE)]

        def wcopy(c, slot):
            return pltpu.make_async_copy(
                w_hbm.at[pl.ds(wbase + c * CH, CH), :],
                w_ring.at[slot], wsems.at[slot])

        def ocopy(c, slot):
            return pltpu.make_async_copy(
                y_ring.at[slot],
                out_hbm.at[pl.ds(obase + c * CH, CH)], osems.at[slot])

        wcopy(0, 0).start()
        wcopy(1, 1).start()

        def pair(i, carry):
            for slot in (0, 1):
                c = 2 * i + slot
                wcopy(c, slot).wait()

                @pl.when(i > 0)
                def _():
                    ocopy(c - 2, slot).wait()

                lane = lax.broadcasted_iota(jnp.int32, (16,), 0)
                for g in range(CH // 16):
                    yvec = jnp.zeros((16,), jnp.float32)
                    for rr in range(16):
                        r = g * 16 + rr
                        accs = [
                            w_ring[slot, r, pl.ds(16 * a, 16)] * evecs[a]
                            for a in range(4)
                        ]
                        for t in range(4, NE, 4):
                            for a in range(4):
                                accs[a] = accs[a] + (
                                    w_ring[slot, r, pl.ds(16 * (t + a), 16)]
                                    * evecs[t + a])
                        acc = (accs[0] + accs[1]) + (accs[2] + accs[3])
                        yvec = jnp.where(lane == rr, jnp.sum(acc), yvec)
                    y_ring[slot, pl.ds(g * 16, 16)] = yvec

                @pl.when(c + 2 < NCH)
                def _():
                    wcopy(c + 2, slot).start()

                ocopy(c, slot).start()
            return carry

        lax.fori_loop(0, NCH // 2, pair, 0)
        ocopy(NCH - 2, 0).wait()
        ocopy(NCH - 1, 1).wait()

    return k(idx_pad, table, W)


def _tc_matvec_body(idx_ref, table_hbm, w_hbm, b_ref, out_ref, m_ref, s_ref,
                    e_rows, e_buf, w_buf, gsem, sems):
    ecopies = [
        pltpu.make_async_copy(
            table_hbm.at[pl.ds(idx_ref[i], 1), :],
            e_rows.at[pl.ds(i, 1), :],
            gsem,
        )
        for i in range(CONTEXT)
    ]
    for c in ecopies:
        c.start()

    def copy_for(blk, nrows):
        slot = lax.rem(blk, DEPTH)
        return pltpu.make_async_copy(
            w_hbm.at[pl.ds(blk * VBS, nrows), :],
            w_buf.at[slot, pl.ds(0, nrows), :],
            sems.at[slot],
        )

    for d in range(DEPTH):
        copy_for(d, VBS).start()

    for c in ecopies:
        c.wait()
    for i in range(CONTEXT):
        e_buf[:, pl.ds(EMBED_DIM * i, EMBED_DIM)] = e_rows[pl.ds(i, 1), :]
    ev = e_buf[...]

    def block_x(blk, nrows):
        slot = lax.rem(blk, DEPTH)
        w = w_buf[slot, pl.ds(0, nrows), :]
        x = lax.dot_general(
            ev, w, (((1,), (1,)), ((), ())),
            preferred_element_type=jnp.float32,
        )
        bb = b_ref[pl.ds(blk * VBS, nrows)].reshape(1, nrows)
        return jnp.maximum(x + bb, 0.0)

    def step(blk, carry):
        m, s = carry
        copy_for(blk, VBS).wait()
        x = block_x(blk, VBS)
        out_ref[:, pl.ds(blk * VBS, VBS)] = x

        @pl.when(blk + DEPTH < NB - 1)
        def _():
            copy_for(blk + DEPTH, VBS).start()

        @pl.when(blk + DEPTH == NB - 1)
        def _():
            copy_for(blk + DEPTH, TAIL).start()

        m_new = jnp.maximum(m, jnp.max(x))
        s_new = s * jnp.exp(m - m_new) + jnp.sum(jnp.exp(x - m_new))
        return m_new, s_new

    m, s = lax.fori_loop(0, NFULL, step, (-jnp.inf, 0.0))

    copy_for(NFULL, TAIL).wait()
    x = block_x(NFULL, TAIL)
    out_ref[:, pl.ds(NFULL * VBS, TAIL)] = x
    m_new = jnp.maximum(m, jnp.max(x))
    s = s * jnp.exp(m - m_new) + jnp.sum(jnp.exp(x - m_new))

    m_ref[0, 0] = m_new
    s_ref[0, 0] = s


def _final_body(lt_ref, m_ref, s_ref, ls_ref, bs_ref, out_ref):
    y = jnp.maximum(ls_ref[...] + bs_ref[...], 0.0).reshape(1, R_SC)
    m_sc = jnp.max(y)
    m_tc = m_ref[0, 0]
    mm = jnp.maximum(m_tc, m_sc)
    s = s_ref[0, 0] * jnp.exp(m_tc - mm) + jnp.sum(jnp.exp(y - mm))
    logz = mm + jnp.log(s)
    out_ref[:, :R0] = lt_ref[...] - logz
    out_ref[:, R0:] = y - logz


def kernel(inputs, emb_table, W, b):
    idx = inputs.astype(jnp.int32)
    idx_pad = jnp.zeros((PAD_CTX,), jnp.int32).at[:CONTEXT].set(idx)

    dots_sc = _sc_matvec(idx_pad, emb_table, W)

    logits_tc, m_tc, s_tc = pl.pallas_call(
        _tc_matvec_body,
        in_specs=[
            pl.BlockSpec(memory_space=pltpu.SMEM),
            pl.BlockSpec(memory_space=pl.ANY),
            pl.BlockSpec(memory_space=pl.ANY),
            pl.BlockSpec((R0,), lambda: (0,)),
        ],
        out_specs=[
            pl.BlockSpec((1, R0), lambda: (0, 0)),
            pl.BlockSpec(memory_space=pltpu.SMEM),
            pl.BlockSpec(memory_space=pltpu.SMEM),
        ],
        out_shape=[
            jax.ShapeDtypeStruct((1, R0), jnp.float32),
            jax.ShapeDtypeStruct((1, 1), jnp.float32),
            jax.ShapeDtypeStruct((1, 1), jnp.float32),
        ],
        scratch_shapes=[
            pltpu.VMEM((CONTEXT, EMBED_DIM), jnp.float32),
            pltpu.VMEM((1, FAN_IN), jnp.float32),
            pltpu.VMEM((DEPTH, VBS, FAN_IN), jnp.float32),
            pltpu.SemaphoreType.DMA,
            pltpu.SemaphoreType.DMA((DEPTH,)),
        ],
    )(idx, emb_table, W, lax.slice(b, [0], [R0]))

    out = pl.pallas_call(
        _final_body,
        in_specs=[
            pl.BlockSpec((1, R0), lambda: (0, 0)),
            pl.BlockSpec(memory_space=pltpu.SMEM),
            pl.BlockSpec(memory_space=pltpu.SMEM),
            pl.BlockSpec((R_SC,), lambda: (0,)),
            pl.BlockSpec((R_SC,), lambda: (0,)),
        ],
        out_specs=pl.BlockSpec((1, VOCAB), lambda: (0, 0)),
        out_shape=jax.ShapeDtypeStruct((1, VOCAB), jnp.float32),
    )(logits_tc, m_tc, s_tc, dots_sc, lax.slice(b, [R0], [VOCAB]))

    return out


# speedup vs baseline: 3.0636x; 3.0636x over previous
import functools

import jax
import jax.numpy as jnp
from jax import lax
from jax.experimental import pallas as pl
from jax.experimental.pallas import tpu as pltpu
from jax.experimental.pallas import tpu_sc as plsc

VOCAB = 100000
EMBED_DIM = 32
CONTEXT = 20
PAD_CTX = 32
FAN_IN = CONTEXT * EMBED_DIM
VB = 5120
NBLK = (VOCAB + VB - 1) // VB
TAIL = VOCAB - (NBLK - 1) * VB


def _sc_gather(idx_pad, table):
    mesh = plsc.VectorSubcoreMesh(core_axis_name="c", subcore_axis_name="s")

    @functools.partial(
        pl.kernel,
        mesh=mesh,
        out_type=jax.ShapeDtypeStruct((PAD_CTX, EMBED_DIM), jnp.float32),
        scratch_types=[
            pltpu.VMEM((PAD_CTX,), jnp.int32),
            pltpu.VMEM((PAD_CTX, EMBED_DIM), jnp.float32),
            pltpu.SemaphoreType.DMA,
        ],
        compiler_params=pltpu.CompilerParams(use_tc_tiling_on_sc=False),
    )
    def k(idx_hbm, table_hbm, out_hbm, idx_v, rows_v, sem):
        wid = lax.axis_index("s") * 2 + lax.axis_index("c")

        @pl.when(wid == 0)
        def _():
            pltpu.sync_copy(idx_hbm, idx_v)
            pltpu.async_copy(table_hbm.at[idx_v], rows_v, sem).wait()
            pltpu.sync_copy(rows_v, out_hbm)

    return k(idx_pad, table)


def _matvec_body(e_ref, w_ref, b_ref, out_ref, m_ref, s_ref):
    i = pl.program_id(0)

    @pl.when(i == 0)
    def _():
        m_ref[0, 0] = -jnp.inf
        s_ref[0, 0] = 0.0

    x = lax.dot_general(
        e_ref[...], w_ref[...], (((1,), (1,)), ((), ())),
        preferred_element_type=jnp.float32,
    )
    x = jnp.maximum(x + b_ref[...].reshape(1, VB), 0.0)

    last = pl.num_programs(0) - 1

    @pl.when(i < last)
    def _():
        out_ref[:, pl.ds(i * VB, VB)] = x

    col = i * VB + lax.broadcasted_iota(jnp.int32, (1, VB), 1)
    xm = jnp.where(col < VOCAB, x, -jnp.inf)
    m_old = m_ref[0, 0]
    m_new = jnp.maximum(m_old, jnp.max(xm))
    s_ref[0, 0] = s_ref[0, 0] * jnp.exp(m_old - m_new) + jnp.sum(
        jnp.exp(xm - m_new))
    m_ref[0, 0] = m_new

    @pl.when(i == last)
    def _():
        out_ref[:, pl.ds(last * VB, TAIL)] = x[:, :TAIL]
        logz = m_ref[0, 0] + jnp.log(s_ref[0, 0])
        out_ref[...] = out_ref[...] - logz


def kernel(inputs, emb_table, W, b):
    idx = jnp.zeros((PAD_CTX,), jnp.int32).at[:CONTEXT].set(
        inputs.astype(jnp.int32))
    rows = _sc_gather(idx, emb_table)
    e = rows[:CONTEXT].reshape(1, FAN_IN)

    out = pl.pallas_call(
        _matvec_body,
        grid=(NBLK,),
        in_specs=[
            pl.BlockSpec((1, FAN_IN), lambda i: (0, 0)),
            pl.BlockSpec((VB, FAN_IN), lambda i: (i, 0)),
            pl.BlockSpec((VB,), lambda i: (i,)),
        ],
        out_specs=pl.BlockSpec((1, VOCAB), lambda i: (0, 0)),
        out_shape=jax.ShapeDtypeStruct((1, VOCAB), jnp.float32),
        scratch_shapes=[
            pltpu.SMEM((1, 1), jnp.float32),
            pltpu.SMEM((1, 1), jnp.float32),
        ],
        compiler_params=pltpu.CompilerParams(
            dimension_semantics=("arbitrary",)),
    )(e, W, b)

    return out


# speedup vs baseline: 3.0726x; 1.0029x over previous
import functools

import jax
import jax.numpy as jnp
from jax import lax
from jax.experimental import pallas as pl
from jax.experimental.pallas import tpu as pltpu
from jax.experimental.pallas import tpu_sc as plsc

VOCAB = 100000
EMBED_DIM = 32
CONTEXT = 20
PAD_CTX = 32
FAN_IN = CONTEXT * EMBED_DIM
VB = 4096
NBLK = (VOCAB + VB - 1) // VB
TAIL = VOCAB - (NBLK - 1) * VB


def _sc_gather(idx_pad, table):
    mesh = plsc.VectorSubcoreMesh(core_axis_name="c", subcore_axis_name="s")

    @functools.partial(
        pl.kernel,
        mesh=mesh,
        out_type=jax.ShapeDtypeStruct((PAD_CTX, EMBED_DIM), jnp.float32),
        scratch_types=[
            pltpu.VMEM((PAD_CTX,), jnp.int32),
            pltpu.VMEM((PAD_CTX, EMBED_DIM), jnp.float32),
            pltpu.SemaphoreType.DMA,
        ],
        compiler_params=pltpu.CompilerParams(use_tc_tiling_on_sc=False),
    )
    def k(idx_hbm, table_hbm, out_hbm, idx_v, rows_v, sem):
        wid = lax.axis_index("s") * 2 + lax.axis_index("c")

        @pl.when(wid == 0)
        def _():
            pltpu.sync_copy(idx_hbm, idx_v)
            pltpu.async_copy(table_hbm.at[idx_v], rows_v, sem).wait()
            pltpu.sync_copy(rows_v, out_hbm)

    return k(idx_pad, table)


def _matvec_body(e_ref, w_ref, b_ref, out_ref, m_ref, s_ref):
    i = pl.program_id(0)

    @pl.when(i == 0)
    def _():
        m_ref[0, 0] = -jnp.inf
        s_ref[0, 0] = 0.0

    x = lax.dot_general(
        e_ref[...], w_ref[...], (((1,), (1,)), ((), ())),
        preferred_element_type=jnp.float32,
    )
    x = jnp.maximum(x + b_ref[...].reshape(1, VB), 0.0)

    last = pl.num_programs(0) - 1

    @pl.when(i < last)
    def _():
        out_ref[:, pl.ds(i * VB, VB)] = x

    col = i * VB + lax.broadcasted_iota(jnp.int32, (1, VB), 1)
    xm = jnp.where(col < VOCAB, x, -jnp.inf)
    m_old = m_ref[0, 0]
    m_new = jnp.maximum(m_old, jnp.max(xm))
    s_ref[0, 0] = s_ref[0, 0] * jnp.exp(m_old - m_new) + jnp.sum(
        jnp.exp(xm - m_new))
    m_ref[0, 0] = m_new

    @pl.when(i == last)
    def _():
        out_ref[:, pl.ds(last * VB, TAIL)] = x[:, :TAIL]
        logz = m_ref[0, 0] + jnp.log(s_ref[0, 0])
        out_ref[...] = out_ref[...] - logz


def kernel(inputs, emb_table, W, b):
    idx = jnp.zeros((PAD_CTX,), jnp.int32).at[:CONTEXT].set(
        inputs.astype(jnp.int32))
    rows = _sc_gather(idx, emb_table)
    e = rows[:CONTEXT].reshape(1, FAN_IN)

    out = pl.pallas_call(
        _matvec_body,
        grid=(NBLK,),
        in_specs=[
            pl.BlockSpec((1, FAN_IN), lambda i: (0, 0)),
            pl.BlockSpec((VB, FAN_IN), lambda i: (i, 0)),
            pl.BlockSpec((VB,), lambda i: (i,)),
        ],
        out_specs=pl.BlockSpec((1, VOCAB), lambda i: (0, 0)),
        out_shape=jax.ShapeDtypeStruct((1, VOCAB), jnp.float32),
        scratch_shapes=[
            pltpu.SMEM((1, 1), jnp.float32),
            pltpu.SMEM((1, 1), jnp.float32),
        ],
        compiler_params=pltpu.CompilerParams(
            dimension_semantics=("arbitrary",)),
    )(e, W, b)

    return out
